# baseline (speedup 1.0000x reference)
import functools

import jax
import jax.numpy as jnp
from jax import lax
from jax.experimental import pallas as pl
from jax.experimental.pallas import tpu as pltpu
from jax.experimental.pallas import tpu_sc as plsc

_N = 10000
_E = 320000
_D = 128
_H = 256
_O = 128

_NC = 2
_NS = 16
_NW = _NC * _NS
_CHUNK = 128
_NCH_TOT = _E // _CHUNK
_NCH = _NCH_TOT // _NW
_NEXTRA = _NCH_TOT - _NCH * _NW
_RPT = (_N // _NS) // 8 * 8
_TAIL = _N - _NS * _RPT

_sc_mesh = plsc.VectorSubcoreMesh(core_axis_name="c", subcore_axis_name="s")


_NR = 3
_NI = 6


@functools.partial(
    pl.kernel,
    out_type=jax.ShapeDtypeStruct((_NC, _N, _D), jnp.float32),
    mesh=_sc_mesh,
    scratch_types=[
        [pltpu.VMEM((_CHUNK,), jnp.int32) for _ in range(_NI)],
        [pltpu.VMEM((_CHUNK,), jnp.int32) for _ in range(_NI)],
        [pltpu.VMEM((_CHUNK, _D), jnp.float32) for _ in range(_NR)],
        pltpu.VMEM_SHARED((_N, _D), jnp.float32),
        [pltpu.SemaphoreType.DMA for _ in range(_NI)],
        [pltpu.SemaphoreType.DMA for _ in range(_NR)],
        [pltpu.SemaphoreType.DMA for _ in range(_NR)],
    ],
)
def _sc_aggregate(x_hbm, zero_hbm, src_hbm, dst_hbm, out_hbm, sidx, didx, rows,
                  accum, semi, semg, sems):
    c = lax.axis_index("c")
    s = lax.axis_index("s")
    r0 = pl.multiple_of(s * _RPT, 8)
    t0 = _NS * _RPT
    wid = c * _NS + s
    first = wid * _NCH

    def _idx_copies(chunk, q):
        off = pl.multiple_of(chunk * _CHUNK, _CHUNK)
        return (
            pltpu.make_async_copy(src_hbm.at[pl.ds(off, _CHUNK)], sidx[q], semi[q]),
            pltpu.make_async_copy(dst_hbm.at[pl.ds(off, _CHUNK)], didx[q], semi[q]),
        )

    def _gather(b, q):
        return pltpu.make_async_copy(x_hbm.at[sidx[q]], rows[b], semg[b])

    def _scatter_start(b, q):
        pltpu.async_copy(rows[b], accum.at[didx[q]], sems[b], add=True)

    def _scatter_wait(b, q):
        pltpu.make_async_copy(rows[b], accum.at[didx[q]], sems[b]).wait()

    for ch in range(4):
        for cp in _idx_copies(first + ch, ch):
            cp.start()

    @pl.when(c == 0)
    def _():
        pltpu.sync_copy(x_hbm.at[pl.ds(r0, _RPT)], accum.at[pl.ds(r0, _RPT)])

        @pl.when(s == _NS - 1)
        def _():
            pltpu.sync_copy(x_hbm.at[pl.ds(t0, _TAIL)], accum.at[pl.ds(t0, _TAIL)])

    @pl.when(c == 1)
    def _():
        pltpu.sync_copy(zero_hbm.at[pl.ds(r0, _RPT)], accum.at[pl.ds(r0, _RPT)])

        @pl.when(s == _NS - 1)
        def _():
            pltpu.sync_copy(zero_hbm.at[pl.ds(t0, _TAIL)],
                            accum.at[pl.ds(t0, _TAIL)])

    plsc.subcore_barrier()
    for cp in _idx_copies(first, 0):
        cp.wait()
    _gather(0, 0).start()

    def body(k, carry):
        for u in range(_NI):
            i = k * _NI + u
            b, q = u % _NR, u
            b1, q1 = (u + 1) % _NR, (u + 1) % _NI

            @pl.when(i + 1 < _NCH)
            def _():
                @pl.when(i >= 2)
                def _():
                    _scatter_wait(b1, (u - 2) % _NI)

                for cp in _idx_copies(first + i + 1, q1):
                    cp.wait()
                _gather(b1, q1).start()

            _gather(b, q).wait()
            _scatter_start(b, q)

            @pl.when(i + 4 < _NCH)
            def _():
                for cp in _idx_copies(first + i + 4, (u + 4) % _NI):
                    cp.start()
        return carry

    lax.fori_loop(0, _NCH // _NI, body, 0)

    for i in range(_NCH - _NR, _NCH):
        _scatter_wait(i % _NR, i % _NI)

    @pl.when(wid < _NEXTRA)
    def _():
        for cp in _idx_copies(_NCH * _NW + wid, 0):
            cp.start()
        for cp in _idx_copies(0, 0):
            cp.wait()
        g = _gather(0, 0)
        g.start()
        g.wait()
        pltpu.sync_copy(rows[0], accum.at[didx[0]], add=True)

    plsc.subcore_barrier()
    pltpu.sync_copy(accum.at[pl.ds(r0, _RPT)], out_hbm.at[c, pl.ds(r0, _RPT)])

    @pl.when(s == _NS - 1)
    def _():
        pltpu.sync_copy(accum.at[pl.ds(t0, _TAIL)], out_hbm.at[c, pl.ds(t0, _TAIL)])


_BLK = 2000


def _mlp_body(p0_ref, p1_ref, w1_ref, b1_ref, w2_ref, b2_ref, w5_ref, z_ref):
    bf = jnp.bfloat16
    s = p0_ref[0] + p1_ref[0]
    h = jnp.maximum(
        jnp.dot(s.astype(bf), w1_ref[...].astype(bf),
                preferred_element_type=jnp.float32) + b1_ref[...], 0.0)
    h = jnp.maximum(
        jnp.dot(h.astype(bf), w2_ref[...].astype(bf),
                preferred_element_type=jnp.float32) + b2_ref[...], 0.0)
    z_ref[...] = jnp.dot(h.astype(bf), w5_ref[...].astype(bf),
                         preferred_element_type=jnp.float32)


def _final_body(q0_ref, q1_ref, b5_ref, o_ref):
    pre = q0_ref[0] + q1_ref[0] + b5_ref[...]
    m = jnp.max(pre, axis=-1, keepdims=True)
    e = jnp.exp(pre - m)
    lse = jnp.log(jnp.sum(e, axis=-1, keepdims=True))
    o_ref[...] = pre - m - lse


def kernel(x, edge_index, W1, b1, W2, b2, W5, b5):
    src = edge_index[0]
    dst = edge_index[1]
    b1r = b1.reshape(1, _H)
    b2r = b2.reshape(1, _H)
    b5r = b5.reshape(1, _O)
    zeros = jnp.zeros((_N, _D), jnp.float32)

    p = _sc_aggregate(x, zeros, src, dst)

    grid = _N // _BLK
    row_spec = lambda d: pl.BlockSpec((_BLK, d), lambda i: (i, 0))
    part_spec = lambda c, d: pl.BlockSpec((1, _BLK, d), lambda i, c=c: (c, i, 0))
    full_spec = lambda r, cdim: pl.BlockSpec((r, cdim), lambda i: (0, 0))

    z = pl.pallas_call(
        _mlp_body,
        grid=(grid,),
        in_specs=[
            part_spec(0, _D), part_spec(1, _D),
            full_spec(_D, _H), full_spec(1, _H),
            full_spec(_H, _H), full_spec(1, _H),
            full_spec(_H, _O),
        ],
        out_specs=row_spec(_O),
        out_shape=jax.ShapeDtypeStruct((_N, _O), jnp.float32),
    )(p, p, W1, b1r, W2, b2r, W5)

    q = _sc_aggregate(z, zeros, src, dst)

    out = pl.pallas_call(
        _final_body,
        grid=(grid,),
        in_specs=[
            part_spec(0, _O), part_spec(1, _O),
            full_spec(1, _O),
        ],
        out_specs=row_spec(_O),
        out_shape=jax.ShapeDtypeStruct((_N, _O), jnp.float32),
    )(q, q, b5r)
    return out


# speedup vs baseline: 1.0079x; 1.0079x over previous
import functools

import jax
import jax.numpy as jnp
from jax import lax
from jax.experimental import pallas as pl
from jax.experimental.pallas import tpu as pltpu
from jax.experimental.pallas import tpu_sc as plsc

_N = 10000
_E = 320000
_D = 128
_H = 256
_O = 128

_NC = 2
_NS = 16
_NW = _NC * _NS
_CHUNK = 128
_NCH_TOT = _E // _CHUNK
_NCH = _NCH_TOT // _NW
_NEXTRA = _NCH_TOT - _NCH * _NW
_RPT = (_N // _NS) // 8 * 8
_TAIL = _N - _NS * _RPT

_sc_mesh = plsc.VectorSubcoreMesh(core_axis_name="c", subcore_axis_name="s")


_NR = 3
_NI = 6


@functools.partial(
    pl.kernel,
    out_type=jax.ShapeDtypeStruct((_NC, _N, _D), jnp.float32),
    mesh=_sc_mesh,
    scratch_types=[
        [pltpu.VMEM((_CHUNK,), jnp.int32) for _ in range(_NI)],
        [pltpu.VMEM((_CHUNK,), jnp.int32) for _ in range(_NI)],
        [pltpu.VMEM((_CHUNK, _D), jnp.float32) for _ in range(_NR)],
        pltpu.VMEM_SHARED((_N, _D), jnp.float32),
        [pltpu.SemaphoreType.DMA for _ in range(_NI)],
        [pltpu.SemaphoreType.DMA for _ in range(_NR)],
        [pltpu.SemaphoreType.DMA for _ in range(_NR)],
        pltpu.VMEM((_CHUNK,), jnp.int32),
        pltpu.VMEM((_CHUNK,), jnp.int32),
        pltpu.SemaphoreType.DMA,
    ],
)
def _sc_aggregate(x_hbm, zero_hbm, src_hbm, dst_hbm, out_hbm, sidx, didx, rows,
                  accum, semi, semg, sems, sidx_e, didx_e, semi_e):
    c = lax.axis_index("c")
    s = lax.axis_index("s")
    r0 = pl.multiple_of(s * _RPT, 8)
    t0 = _NS * _RPT
    wid = c * _NS + s
    first = wid * _NCH

    def _idx_copies(chunk, q):
        off = pl.multiple_of(chunk * _CHUNK, _CHUNK)
        return (
            pltpu.make_async_copy(src_hbm.at[pl.ds(off, _CHUNK)], sidx[q], semi[q]),
            pltpu.make_async_copy(dst_hbm.at[pl.ds(off, _CHUNK)], didx[q], semi[q]),
        )

    def _gather(b, q):
        return pltpu.make_async_copy(x_hbm.at[sidx[q]], rows[b], semg[b])

    def _scatter_start(b, q):
        pltpu.async_copy(rows[b], accum.at[didx[q]], sems[b], add=True)

    def _scatter_wait(b, q):
        pltpu.make_async_copy(rows[b], accum.at[didx[q]], sems[b]).wait()

    def _extra_copies():
        off = pl.multiple_of((_NCH * _NW + wid) * _CHUNK, _CHUNK)
        return (
            pltpu.make_async_copy(src_hbm.at[pl.ds(off, _CHUNK)], sidx_e, semi_e),
            pltpu.make_async_copy(dst_hbm.at[pl.ds(off, _CHUNK)], didx_e, semi_e),
        )

    def _gather_e():
        return pltpu.make_async_copy(x_hbm.at[sidx_e], rows[0], semg[0])

    for ch in range(4):
        for cp in _idx_copies(first + ch, ch):
            cp.start()

    @pl.when(wid < _NEXTRA)
    def _():
        for cp in _extra_copies():
            cp.start()

    @pl.when(c == 0)
    def _():
        pltpu.sync_copy(x_hbm.at[pl.ds(r0, _RPT)], accum.at[pl.ds(r0, _RPT)])

        @pl.when(s == _NS - 1)
        def _():
            pltpu.sync_copy(x_hbm.at[pl.ds(t0, _TAIL)], accum.at[pl.ds(t0, _TAIL)])

    @pl.when(c == 1)
    def _():
        pltpu.sync_copy(zero_hbm.at[pl.ds(r0, _RPT)], accum.at[pl.ds(r0, _RPT)])

        @pl.when(s == _NS - 1)
        def _():
            pltpu.sync_copy(zero_hbm.at[pl.ds(t0, _TAIL)],
                            accum.at[pl.ds(t0, _TAIL)])

    plsc.subcore_barrier()
    for cp in _idx_copies(first, 0):
        cp.wait()
    _gather(0, 0).start()

    def body(k, carry):
        for u in range(_NI):
            i = k * _NI + u
            b, q = u % _NR, u
            b1, q1 = (u + 1) % _NR, (u + 1) % _NI

            @pl.when(i + 1 < _NCH)
            def _():
                @pl.when(i >= 2)
                def _():
                    _scatter_wait(b1, (u - 2) % _NI)

                for cp in _idx_copies(first + i + 1, q1):
                    cp.wait()
                _gather(b1, q1).start()

            _gather(b, q).wait()
            _scatter_start(b, q)

            @pl.when(i + 4 < _NCH)
            def _():
                for cp in _idx_copies(first + i + 4, (u + 4) % _NI):
                    cp.start()
        return carry

    lax.fori_loop(0, _NCH // _NI, body, 0)

    for i in range(_NCH - _NR, _NCH):
        _scatter_wait(i % _NR, i % _NI)

    @pl.when(wid < _NEXTRA)
    def _():
        for cp in _extra_copies():
            cp.wait()
        g = _gather_e()
        g.start()
        g.wait()
        pltpu.sync_copy(rows[0], accum.at[didx_e], add=True)

    plsc.subcore_barrier()
    pltpu.sync_copy(accum.at[pl.ds(r0, _RPT)], out_hbm.at[c, pl.ds(r0, _RPT)])

    @pl.when(s == _NS - 1)
    def _():
        pltpu.sync_copy(accum.at[pl.ds(t0, _TAIL)], out_hbm.at[c, pl.ds(t0, _TAIL)])


_BLK = 2000


def _mlp_body(p0_ref, p1_ref, w1_ref, b1_ref, w2_ref, b2_ref, w5_ref, z_ref):
    bf = jnp.bfloat16
    s = p0_ref[0] + p1_ref[0]
    h = jnp.maximum(
        jnp.dot(s.astype(bf), w1_ref[...].astype(bf),
                preferred_element_type=jnp.float32) + b1_ref[...], 0.0)
    h = jnp.maximum(
        jnp.dot(h.astype(bf), w2_ref[...].astype(bf),
                preferred_element_type=jnp.float32) + b2_ref[...], 0.0)
    z_ref[...] = jnp.dot(h.astype(bf), w5_ref[...].astype(bf),
                         preferred_element_type=jnp.float32)


def _final_body(q0_ref, q1_ref, b5_ref, o_ref):
    pre = q0_ref[0] + q1_ref[0] + b5_ref[...]
    m = jnp.max(pre, axis=-1, keepdims=True)
    e = jnp.exp(pre - m)
    lse = jnp.log(jnp.sum(e, axis=-1, keepdims=True))
    o_ref[...] = pre - m - lse


def kernel(x, edge_index, W1, b1, W2, b2, W5, b5):
    src = edge_index[0]
    dst = edge_index[1]
    b1r = b1.reshape(1, _H)
    b2r = b2.reshape(1, _H)
    b5r = b5.reshape(1, _O)
    zeros = jnp.zeros((_N, _D), jnp.float32)

    p = _sc_aggregate(x, zeros, src, dst)

    grid = _N // _BLK
    row_spec = lambda d: pl.BlockSpec((_BLK, d), lambda i: (i, 0))
    part_spec = lambda c, d: pl.BlockSpec((1, _BLK, d), lambda i, c=c: (c, i, 0))
    full_spec = lambda r, cdim: pl.BlockSpec((r, cdim), lambda i: (0, 0))

    z = pl.pallas_call(
        _mlp_body,
        grid=(grid,),
        in_specs=[
            part_spec(0, _D), part_spec(1, _D),
            full_spec(_D, _H), full_spec(1, _H),
            full_spec(_H, _H), full_spec(1, _H),
            full_spec(_H, _O),
        ],
        out_specs=row_spec(_O),
        out_shape=jax.ShapeDtypeStruct((_N, _O), jnp.float32),
    )(p, p, W1, b1r, W2, b2r, W5)

    q = _sc_aggregate(z, zeros, src, dst)

    out = pl.pallas_call(
        _final_body,
        grid=(grid,),
        in_specs=[
            part_spec(0, _O), part_spec(1, _O),
            full_spec(1, _O),
        ],
        out_specs=row_spec(_O),
        out_shape=jax.ShapeDtypeStruct((_N, _O), jnp.float32),
    )(q, q, b5r)
    return out


# speedup vs baseline: 1.0140x; 1.0060x over previous
import functools

import jax
import jax.numpy as jnp
from jax import lax
from jax.experimental import pallas as pl
from jax.experimental.pallas import tpu as pltpu
from jax.experimental.pallas import tpu_sc as plsc

_N = 10000
_E = 320000
_D = 128
_H = 256
_O = 128

_NC = 2
_NS = 16
_NW = _NC * _NS
_CHUNK = 128
_NCH_TOT = _E // _CHUNK
_NCH = _NCH_TOT // _NW
_NEXTRA = _NCH_TOT - _NCH * _NW
_RPT = (_N // _NS) // 8 * 8
_TAIL = _N - _NS * _RPT

_sc_mesh = plsc.VectorSubcoreMesh(core_axis_name="c", subcore_axis_name="s")


_NR = 3
_NI = 6


@functools.partial(
    pl.kernel,
    out_type=jax.ShapeDtypeStruct((_NC, _N, _D), jnp.float32),
    mesh=_sc_mesh,
    scratch_types=[
        [pltpu.VMEM((_CHUNK,), jnp.int32) for _ in range(_NI)],
        [pltpu.VMEM((_CHUNK,), jnp.int32) for _ in range(_NI)],
        [pltpu.VMEM((_CHUNK, _D), jnp.float32) for _ in range(_NR)],
        pltpu.VMEM_SHARED((_N, _D), jnp.float32),
        [pltpu.SemaphoreType.DMA for _ in range(_NI)],
        [pltpu.SemaphoreType.DMA for _ in range(_NR)],
        [pltpu.SemaphoreType.DMA for _ in range(_NR)],
        pltpu.VMEM((_CHUNK,), jnp.int32),
        pltpu.VMEM((_CHUNK,), jnp.int32),
        pltpu.SemaphoreType.DMA,
    ],
)
def _sc_aggregate(x_hbm, zero_hbm, src_hbm, dst_hbm, out_hbm, sidx, didx, rows,
                  accum, semi, semg, sems, sidx_e, didx_e, semi_e):
    c = lax.axis_index("c")
    s = lax.axis_index("s")
    r0 = pl.multiple_of(s * _RPT, 8)
    t0 = _NS * _RPT
    wid = c * _NS + s
    first = wid * _NCH

    def _idx_copies(chunk, q):
        off = pl.multiple_of(chunk * _CHUNK, _CHUNK)
        return (
            pltpu.make_async_copy(src_hbm.at[pl.ds(off, _CHUNK)], sidx[q], semi[q]),
            pltpu.make_async_copy(dst_hbm.at[pl.ds(off, _CHUNK)], didx[q], semi[q]),
        )

    def _gather(b, q):
        return pltpu.make_async_copy(x_hbm.at[sidx[q]], rows[b], semg[b])

    def _scatter_start(b, q):
        pltpu.async_copy(rows[b], accum.at[didx[q]], sems[b], add=True)

    def _scatter_wait(b, q):
        pltpu.make_async_copy(rows[b], accum.at[didx[q]], sems[b]).wait()

    def _extra_copies():
        off = pl.multiple_of((_NCH * _NW + wid) * _CHUNK, _CHUNK)
        return (
            pltpu.make_async_copy(src_hbm.at[pl.ds(off, _CHUNK)], sidx_e, semi_e),
            pltpu.make_async_copy(dst_hbm.at[pl.ds(off, _CHUNK)], didx_e, semi_e),
        )

    def _gather_e():
        return pltpu.make_async_copy(x_hbm.at[sidx_e], rows[0], semg[0])

    for ch in range(4):
        for cp in _idx_copies(first + ch, ch):
            cp.start()

    @pl.when(wid < _NEXTRA)
    def _():
        for cp in _extra_copies():
            cp.start()

    @pl.when(c == 0)
    def _():
        pltpu.sync_copy(x_hbm.at[pl.ds(r0, _RPT)], accum.at[pl.ds(r0, _RPT)])

        @pl.when(s == _NS - 1)
        def _():
            pltpu.sync_copy(x_hbm.at[pl.ds(t0, _TAIL)], accum.at[pl.ds(t0, _TAIL)])

    @pl.when(c == 1)
    def _():
        pltpu.sync_copy(zero_hbm.at[pl.ds(r0, _RPT)], accum.at[pl.ds(r0, _RPT)])

        @pl.when(s == _NS - 1)
        def _():
            pltpu.sync_copy(zero_hbm.at[pl.ds(t0, _TAIL)],
                            accum.at[pl.ds(t0, _TAIL)])

    for cp in _idx_copies(first, 0):
        cp.wait()
    _gather(0, 0).start()
    plsc.subcore_barrier()

    def body(k, carry):
        for u in range(_NI):
            i = k * _NI + u
            b, q = u % _NR, u
            b1, q1 = (u + 1) % _NR, (u + 1) % _NI

            @pl.when(i + 1 < _NCH)
            def _():
                @pl.when(i >= 2)
                def _():
                    _scatter_wait(b1, (u - 2) % _NI)

                for cp in _idx_copies(first + i + 1, q1):
                    cp.wait()
                _gather(b1, q1).start()

            _gather(b, q).wait()
            _scatter_start(b, q)

            @pl.when(i + 4 < _NCH)
            def _():
                for cp in _idx_copies(first + i + 4, (u + 4) % _NI):
                    cp.start()
        return carry

    lax.fori_loop(0, _NCH // _NI, body, 0)

    for i in range(_NCH - _NR, _NCH):
        _scatter_wait(i % _NR, i % _NI)

    @pl.when(wid < _NEXTRA)
    def _():
        for cp in _extra_copies():
            cp.wait()
        g = _gather_e()
        g.start()
        g.wait()
        pltpu.sync_copy(rows[0], accum.at[didx_e], add=True)

    plsc.subcore_barrier()
    pltpu.sync_copy(accum.at[pl.ds(r0, _RPT)], out_hbm.at[c, pl.ds(r0, _RPT)])

    @pl.when(s == _NS - 1)
    def _():
        pltpu.sync_copy(accum.at[pl.ds(t0, _TAIL)], out_hbm.at[c, pl.ds(t0, _TAIL)])


_BLK = 2000


def _mlp_body(p0_ref, p1_ref, w1_ref, b1_ref, w2_ref, b2_ref, w5_ref, z_ref):
    bf = jnp.bfloat16
    s = p0_ref[0] + p1_ref[0]
    h = jnp.maximum(
        jnp.dot(s.astype(bf), w1_ref[...].astype(bf),
                preferred_element_type=jnp.float32) + b1_ref[...], 0.0)
    h = jnp.maximum(
        jnp.dot(h.astype(bf), w2_ref[...].astype(bf),
                preferred_element_type=jnp.float32) + b2_ref[...], 0.0)
    z_ref[...] = jnp.dot(h.astype(bf), w5_ref[...].astype(bf),
                         preferred_element_type=jnp.float32)


def _final_body(q0_ref, q1_ref, b5_ref, o_ref):
    pre = q0_ref[0] + q1_ref[0] + b5_ref[...]
    m = jnp.max(pre, axis=-1, keepdims=True)
    e = jnp.exp(pre - m)
    lse = jnp.log(jnp.sum(e, axis=-1, keepdims=True))
    o_ref[...] = pre - m - lse


def kernel(x, edge_index, W1, b1, W2, b2, W5, b5):
    src = edge_index[0]
    dst = edge_index[1]
    b1r = b1.reshape(1, _H)
    b2r = b2.reshape(1, _H)
    b5r = b5.reshape(1, _O)
    zeros = jnp.zeros((_N, _D), jnp.float32)

    p = _sc_aggregate(x, zeros, src, dst)

    grid = _N // _BLK
    row_spec = lambda d: pl.BlockSpec((_BLK, d), lambda i: (i, 0))
    part_spec = lambda c, d: pl.BlockSpec((1, _BLK, d), lambda i, c=c: (c, i, 0))
    full_spec = lambda r, cdim: pl.BlockSpec((r, cdim), lambda i: (0, 0))

    z = pl.pallas_call(
        _mlp_body,
        grid=(grid,),
        in_specs=[
            part_spec(0, _D), part_spec(1, _D),
            full_spec(_D, _H), full_spec(1, _H),
            full_spec(_H, _H), full_spec(1, _H),
            full_spec(_H, _O),
        ],
        out_specs=row_spec(_O),
        out_shape=jax.ShapeDtypeStruct((_N, _O), jnp.float32),
    )(p, p, W1, b1r, W2, b2r, W5)

    q = _sc_aggregate(z, zeros, src, dst)

    out = pl.pallas_call(
        _final_body,
        grid=(grid,),
        in_specs=[
            part_spec(0, _O), part_spec(1, _O),
            full_spec(1, _O),
        ],
        out_specs=row_spec(_O),
        out_shape=jax.ShapeDtypeStruct((_N, _O), jnp.float32),
    )(q, q, b5r)
    return out
